# Initial kernel scaffold; baseline (speedup 1.0000x reference)
import jax
import jax.numpy as jnp
from jax.experimental import pallas as pl


def kernel(grade, imgs):
    raise NotImplementedError("write your pallas kernel here")


# speedup vs baseline: 1.5163x; 1.5163x over previous
import jax
import jax.numpy as jnp
from jax.experimental import pallas as pl

_G0, _G1, _D = 32, 32, 256
_N = _G0 * _G1
_B = 512
_LR = 0.1
_SIGMA = 2.0


def _som_body(gT_ref, imgsT_ref, w_ref, out_ref):
    out_ref[:, :] = gT_ref[:, :]
    lane_iota = jax.lax.broadcasted_iota(jnp.int32, (1, _N), 1)
    img_iota = jax.lax.broadcasted_iota(jnp.int32, (_B, 1), 0)

    def step(t, carry):
        g = out_ref[:, :]
        oh_t = (img_iota == t).astype(jnp.float32)
        img = jnp.dot(imgsT_ref[:, :], oh_t,
                      precision=jax.lax.Precision.HIGHEST,
                      preferred_element_type=jnp.float32)
        diff = g - img
        d2 = jnp.sum(diff * diff, axis=0, keepdims=True)
        m = jnp.min(d2)
        k = jnp.min(jnp.where(d2 == m, lane_iota, _N))
        oh_w = (lane_iota == k).astype(jnp.float32)
        a = jnp.dot(oh_w, w_ref[:, :],
                    precision=jax.lax.Precision.HIGHEST,
                    preferred_element_type=jnp.float32)
        out_ref[:, :] = g - a * diff
        return carry

    jax.lax.fori_loop(0, _B, step, 0)


def kernel(grade, imgs):
    gT = grade.reshape(_N, _D).T
    imgsT = imgs.T
    k1 = jnp.arange(_N, dtype=jnp.int32)
    i1 = (k1 // _G1).astype(jnp.float32)
    j1 = (k1 % _G1).astype(jnp.float32)
    di = i1[:, None] - i1[None, :]
    dj = j1[:, None] - j1[None, :]
    d = jnp.sqrt(di * di + dj * dj)
    w = jnp.float32(_LR) * jnp.exp(-(d * d) / (2.0 * jnp.float32(_SIGMA) ** 2))
    outT = pl.pallas_call(
        _som_body,
        out_shape=jax.ShapeDtypeStruct((_D, _N), jnp.float32),
    )(gT, imgsT, w)
    return outT.T.reshape(_G0, _G1, _D)


# speedup vs baseline: 5.6444x; 3.7225x over previous
import jax
import jax.numpy as jnp
from jax.experimental import pallas as pl

_G0, _G1, _D = 32, 32, 256
_N = _G0 * _G1
_B = 512
_LR = 0.1
_SIGMA = 2.0


def _som_body(gT_ref, imgs_ref, w_ref, out_ref):
    out_ref[:, :] = gT_ref[:, :]
    lane_iota = jax.lax.broadcasted_iota(jnp.int32, (1, _N), 1)

    def step(t, carry):
        g = out_ref[:, :]
        img = imgs_ref[pl.ds(t, 1), :].T
        diff = g - img
        d2 = jnp.sum(diff * diff, axis=0, keepdims=True)
        m = jnp.min(d2)
        k = jnp.min(jnp.where(d2 == m, lane_iota, _N))
        a = w_ref[pl.ds(k, 1), :]
        out_ref[:, :] = g - a * diff
        return carry

    jax.lax.fori_loop(0, _B, step, 0)


def kernel(grade, imgs):
    gT = grade.reshape(_N, _D).T
    k1 = jnp.arange(_N, dtype=jnp.int32)
    i1 = (k1 // _G1).astype(jnp.float32)
    j1 = (k1 % _G1).astype(jnp.float32)
    di = i1[:, None] - i1[None, :]
    dj = j1[:, None] - j1[None, :]
    d = jnp.sqrt(di * di + dj * dj)
    w = jnp.float32(_LR) * jnp.exp(-(d * d) / (2.0 * jnp.float32(_SIGMA) ** 2))
    outT = pl.pallas_call(
        _som_body,
        out_shape=jax.ShapeDtypeStruct((_D, _N), jnp.float32),
    )(gT, imgs, w)
    return outT.T.reshape(_G0, _G1, _D)


# speedup vs baseline: 7.2401x; 1.2827x over previous
import jax
import jax.numpy as jnp
from jax.experimental import pallas as pl

_G0, _G1, _D = 32, 32, 256
_N = _G0 * _G1
_B = 512
_LR = 0.1
_SIGMA = 2.0


def _som_body(gT_ref, imgs_ref, w_ref, out_ref):
    out_ref[:, :] = gT_ref[:, :]
    lane_iota = jax.lax.broadcasted_iota(jnp.int32, (1, _N), 1)

    img0 = imgs_ref[pl.ds(0, 1), :].T
    diff0 = out_ref[:, :] - img0
    d2_0 = jnp.sum(diff0 * diff0, axis=0, keepdims=True)

    def step(t, carry):
        d2, img = carry
        m = jnp.min(d2)
        k = jnp.min(jnp.where(d2 == m, lane_iota, _N))
        a = w_ref[pl.ds(k, 1), :]
        g = out_ref[:, :]
        diff = g - img
        gp = g - a * diff
        out_ref[:, :] = gp
        tn = jnp.minimum(t + 1, _B - 1)
        imgn = imgs_ref[pl.ds(tn, 1), :].T
        diffn = gp - imgn
        d2n = jnp.sum(diffn * diffn, axis=0, keepdims=True)
        return (d2n, imgn)

    jax.lax.fori_loop(0, _B, step, (d2_0, img0), unroll=2)


def kernel(grade, imgs):
    gT = grade.reshape(_N, _D).T
    k1 = jnp.arange(_N, dtype=jnp.int32)
    i1 = (k1 // _G1).astype(jnp.float32)
    j1 = (k1 % _G1).astype(jnp.float32)
    di = i1[:, None] - i1[None, :]
    dj = j1[:, None] - j1[None, :]
    d = jnp.sqrt(di * di + dj * dj)
    w = jnp.float32(_LR) * jnp.exp(-(d * d) / (2.0 * jnp.float32(_SIGMA) ** 2))
    outT = pl.pallas_call(
        _som_body,
        out_shape=jax.ShapeDtypeStruct((_D, _N), jnp.float32),
    )(gT, imgs, w)
    return outT.T.reshape(_G0, _G1, _D)
